# baseline (speedup 1.0000x reference)
import functools

import numpy as np
import jax
import jax.numpy as jnp
from jax import lax
from jax.experimental import pallas as pl
from jax.experimental.pallas import tpu as pltpu
from jax.experimental.pallas import tpu_sc as plsc

_MARGIN = 1.0
_MAX_PAIRS = 2000000
_NC, _NS, _L = 2, 16, 16
_NW = _NC * _NS
_B = 4096


_pair_cache = {}


def _rotl(x, d):
    return ((x << np.uint32(d)) | (x >> np.uint32(32 - d))).astype(np.uint32)


def _threefry2x32(keypair, x0, x1):
    x0 = np.asarray(x0, np.uint32).copy()
    x1 = np.asarray(x1, np.uint32).copy()
    ks0 = np.uint32(keypair[0])
    ks1 = np.uint32(keypair[1])
    ks2 = np.uint32(ks0 ^ ks1 ^ np.uint32(0x1BD11BDA))
    rot0 = (13, 15, 26, 6)
    rot1 = (17, 29, 16, 24)

    def rounds(x0, x1, rots):
        for r in rots:
            x0 = (x0 + x1).astype(np.uint32)
            x1 = _rotl(x1, r)
            x1 = x1 ^ x0
        return x0, x1

    x0 = (x0 + ks0).astype(np.uint32)
    x1 = (x1 + ks1).astype(np.uint32)
    x0, x1 = rounds(x0, x1, rot0)
    x0 = (x0 + ks1).astype(np.uint32)
    x1 = (x1 + ks2 + np.uint32(1)).astype(np.uint32)
    x0, x1 = rounds(x0, x1, rot1)
    x0 = (x0 + ks2).astype(np.uint32)
    x1 = (x1 + ks0 + np.uint32(2)).astype(np.uint32)
    x0, x1 = rounds(x0, x1, rot0)
    x0 = (x0 + ks0).astype(np.uint32)
    x1 = (x1 + ks1 + np.uint32(3)).astype(np.uint32)
    x0, x1 = rounds(x0, x1, rot1)
    x0 = (x0 + ks1).astype(np.uint32)
    x1 = (x1 + ks2 + np.uint32(4)).astype(np.uint32)
    x0, x1 = rounds(x0, x1, rot0)
    x0 = (x0 + ks2).astype(np.uint32)
    x1 = (x1 + ks0 + np.uint32(5)).astype(np.uint32)
    return x0, x1


def _np_split(keypair, num=2):
    counts = np.arange(num, dtype=np.uint64)
    b1, b2 = _threefry2x32(keypair, (counts >> np.uint64(32)).astype(np.uint32),
                           (counts & np.uint64(0xFFFFFFFF)).astype(np.uint32))
    return np.stack([b1, b2], axis=1)


def _np_random_bits(keypair, size):
    counts = np.arange(size, dtype=np.uint64)
    b1, b2 = _threefry2x32(keypair, (counts >> np.uint64(32)).astype(np.uint32),
                           (counts & np.uint64(0xFFFFFFFF)).astype(np.uint32))
    return b1 ^ b2


def _np_randint(keypair, size, minval, maxval):
    khi, klo = _np_split(keypair, 2)
    higher = _np_random_bits(khi, size)
    lower = _np_random_bits(klo, size)
    span = np.uint32(maxval - minval)
    multiplier = np.uint32((int(np.uint32(65536) % span) ** 2) & 0xFFFFFFFF) % span
    with np.errstate(over="ignore"):
        offset = ((higher % span) * multiplier + (lower % span)) % span
    return (np.int32(minval) + offset.astype(np.int32)).astype(np.int32)


def _pair_layout(n):
    if n in _pair_cache:
        return _pair_cache[n]
    n_pairs = min(_MAX_PAIRS, n * (n - 1) // 2)
    root = np.array([0, 42], np.uint32)
    ki, kj = _np_split(root, 2)
    idx_i = _np_randint(ki, n_pairs, 0, n)
    idx_j = _np_randint(kj, n_pairs, 0, n)
    keep = idx_i != idx_j
    idx_i, idx_j = idx_i[keep], idx_j[keep]
    m = idx_i.shape[0]
    nch = -(-(-(-m // _NW)) // _B)
    c_tile = nch * _B
    total = c_tile * _NW
    ii = np.zeros((total,), np.int32)
    jj = np.zeros((total,), np.int32)
    ii[:m] = idx_i
    jj[:m] = idx_j
    idx = np.stack([ii.reshape(_NW, nch, _B), jj.reshape(_NW, nch, _B)], axis=2)
    out = (jnp.asarray(idx), nch, c_tile)
    _pair_cache[n] = out
    return out


def _final_kernel(lp, cp, o):
    s = jnp.sum(lp[...])
    c = jnp.sum(cp[...])
    o[0] = s / jnp.maximum(c, 1.0)


def _pack_kernel(c0, c1, c2, c3, pr, o):
    t0 = 1.0 - 0.25 * jnp.clip(c0[...], 0.0, 1.0)
    t1 = 1.0 - 0.25 * jnp.clip(c1[...], 0.0, 1.0)
    t2 = 1.0 - 0.25 * jnp.clip(c2[...], 0.0, 1.0)
    t3 = 1.0 - 0.25 * jnp.clip(c3[...], 0.0, 1.0)
    ct = 1.0 - t0 * t1 * t2 * t3
    pb = lax.bitcast_convert_type(pr[...].astype(jnp.bfloat16), jnp.uint16)
    cb = lax.bitcast_convert_type(ct.astype(jnp.bfloat16), jnp.uint16)
    word = (cb.astype(jnp.uint32) << 16) | pb.astype(jnp.uint32)
    o[...] = lax.bitcast_convert_type(word, jnp.float32)


def _make_sc_loss(n, nch, c_tile):
    mesh = plsc.VectorSubcoreMesh(core_axis_name="c", subcore_axis_name="s")

    @functools.partial(
        pl.kernel,
        out_type=[
            jax.ShapeDtypeStruct((_NW, _L), jnp.float32),
            jax.ShapeDtypeStruct((_NW, _L), jnp.float32),
        ],
        mesh=mesh,
        compiler_params=pltpu.CompilerParams(
            needs_layout_passes=False, use_tc_tiling_on_sc=False),
        scratch_types=[
            pltpu.VMEM((n,), jnp.float32),
            pltpu.VMEM((2, 2, _B), jnp.int32),
            pltpu.VMEM((2 * _L,), jnp.float32),
            pltpu.SemaphoreType.DMA((2,)),
            pltpu.SemaphoreType.DMA,
        ],
    )
    def sc_loss(tab_hbm, idx_hbm, loss_out, cnt_out, table, idxb, pout, isems,
                tsem):
        cid = lax.axis_index("c")
        sid = lax.axis_index("s")
        wid = sid * _NC + cid
        nvec = _B // _L

        in_d = [None] * nch
        in_d[0] = pltpu.async_copy(idx_hbm.at[wid, 0], idxb.at[0], isems.at[0])
        pltpu.async_copy(tab_hbm.at[pl.ds(0, n)], table, tsem).wait()
        acc = (jnp.zeros((_L,), jnp.float32), jnp.zeros((_L,), jnp.float32))
        for ch in range(nch):
            cur = ch % 2
            if ch + 1 < nch:
                in_d[ch + 1] = pltpu.async_copy(
                    idx_hbm.at[wid, ch + 1], idxb.at[1 - cur], isems.at[1 - cur])
            in_d[ch].wait()

            def p2(v, carry):
                al, ac = carry
                off = pl.multiple_of(v * _L, _L)
                ii = idxb[cur, 0, pl.ds(off, _L)]
                jj = idxb[cur, 1, pl.ds(off, _L)]
                gi = plsc.load_gather(table, [ii])
                gj = plsc.load_gather(table, [jj])
                pi, ci = plsc.unpack(plsc.bitcast(gi, jnp.bfloat16),
                                     format=plsc.PackFormat.INTERLEAVED)
                pj, cj = plsc.unpack(plsc.bitcast(gj, jnp.bfloat16),
                                     format=plsc.PackFormat.INTERLEAVED)
                pd = pi - pj
                cd = ci - cj
                sbit = plsc.bitcast(cd, jnp.int32) & jnp.int32(-2147483648)
                pdx = plsc.bitcast(plsc.bitcast(pd, jnp.int32) ^ sbit,
                                   jnp.float32)
                elem = jnp.maximum(_MARGIN - pdx, 0.0)
                mf = jnp.where(jnp.abs(cd) > 0.1, 1.0, 0.0)
                return (al + elem * mf, ac + mf)

            acc = plsc.parallel_loop(0, nvec, step=1, unroll=8, carry=acc)(p2)

        pout[pl.ds(0, _L)] = acc[0]
        pout[pl.ds(_L, _L)] = acc[1]
        pltpu.sync_copy(pout.at[pl.ds(0, _L)], loss_out.at[wid])
        pltpu.sync_copy(pout.at[pl.ds(_L, _L)], cnt_out.at[wid])

    return sc_loss


def kernel(pred_scores, cic_scores):
    pred = pred_scores.reshape(-1).astype(jnp.float32)
    n = pred.shape[0]
    idx, nch, c_tile = _pair_layout(n)

    npad = -(-n // 128) * 128
    rows = npad // 128
    cic_t = jnp.pad(cic_scores.astype(jnp.float32), ((0, npad - n), (0, 0))).T
    cols = cic_t.reshape(4, rows, 128)
    pred_rows = jnp.pad(pred, (0, npad - n)).reshape(rows, 128)
    packed = pl.pallas_call(
        _pack_kernel,
        out_shape=jax.ShapeDtypeStruct((rows, 128), jnp.float32),
    )(cols[0], cols[1], cols[2], cols[3], pred_rows).reshape(npad)

    sc_loss = _make_sc_loss(n, nch, c_tile)
    loss_part, cnt_part = sc_loss(packed, idx)

    out = pl.pallas_call(
        _final_kernel,
        out_shape=jax.ShapeDtypeStruct((1,), jnp.float32),
        out_specs=pl.BlockSpec(memory_space=pltpu.SMEM),
    )(loss_part, cnt_part)
    return out[0]


# speedup vs baseline: 1.0050x; 1.0050x over previous
import functools

import numpy as np
import jax
import jax.numpy as jnp
from jax import lax
from jax.experimental import pallas as pl
from jax.experimental.pallas import tpu as pltpu
from jax.experimental.pallas import tpu_sc as plsc

_MARGIN = 1.0
_MAX_PAIRS = 2000000
_NC, _NS, _L = 2, 16, 16
_NW = _NC * _NS
_B = 4096


_pair_cache = {}


def _rotl(x, d):
    return ((x << np.uint32(d)) | (x >> np.uint32(32 - d))).astype(np.uint32)


def _threefry2x32(keypair, x0, x1):
    x0 = np.asarray(x0, np.uint32).copy()
    x1 = np.asarray(x1, np.uint32).copy()
    ks0 = np.uint32(keypair[0])
    ks1 = np.uint32(keypair[1])
    ks2 = np.uint32(ks0 ^ ks1 ^ np.uint32(0x1BD11BDA))
    rot0 = (13, 15, 26, 6)
    rot1 = (17, 29, 16, 24)

    def rounds(x0, x1, rots):
        for r in rots:
            x0 = (x0 + x1).astype(np.uint32)
            x1 = _rotl(x1, r)
            x1 = x1 ^ x0
        return x0, x1

    x0 = (x0 + ks0).astype(np.uint32)
    x1 = (x1 + ks1).astype(np.uint32)
    x0, x1 = rounds(x0, x1, rot0)
    x0 = (x0 + ks1).astype(np.uint32)
    x1 = (x1 + ks2 + np.uint32(1)).astype(np.uint32)
    x0, x1 = rounds(x0, x1, rot1)
    x0 = (x0 + ks2).astype(np.uint32)
    x1 = (x1 + ks0 + np.uint32(2)).astype(np.uint32)
    x0, x1 = rounds(x0, x1, rot0)
    x0 = (x0 + ks0).astype(np.uint32)
    x1 = (x1 + ks1 + np.uint32(3)).astype(np.uint32)
    x0, x1 = rounds(x0, x1, rot1)
    x0 = (x0 + ks1).astype(np.uint32)
    x1 = (x1 + ks2 + np.uint32(4)).astype(np.uint32)
    x0, x1 = rounds(x0, x1, rot0)
    x0 = (x0 + ks2).astype(np.uint32)
    x1 = (x1 + ks0 + np.uint32(5)).astype(np.uint32)
    return x0, x1


def _np_split(keypair, num=2):
    counts = np.arange(num, dtype=np.uint64)
    b1, b2 = _threefry2x32(keypair, (counts >> np.uint64(32)).astype(np.uint32),
                           (counts & np.uint64(0xFFFFFFFF)).astype(np.uint32))
    return np.stack([b1, b2], axis=1)


def _np_random_bits(keypair, size):
    counts = np.arange(size, dtype=np.uint64)
    b1, b2 = _threefry2x32(keypair, (counts >> np.uint64(32)).astype(np.uint32),
                           (counts & np.uint64(0xFFFFFFFF)).astype(np.uint32))
    return b1 ^ b2


def _np_randint(keypair, size, minval, maxval):
    khi, klo = _np_split(keypair, 2)
    higher = _np_random_bits(khi, size)
    lower = _np_random_bits(klo, size)
    span = np.uint32(maxval - minval)
    multiplier = np.uint32((int(np.uint32(65536) % span) ** 2) & 0xFFFFFFFF) % span
    with np.errstate(over="ignore"):
        offset = ((higher % span) * multiplier + (lower % span)) % span
    return (np.int32(minval) + offset.astype(np.int32)).astype(np.int32)


def _pair_layout(n):
    if n in _pair_cache:
        return _pair_cache[n]
    n_pairs = min(_MAX_PAIRS, n * (n - 1) // 2)
    root = np.array([0, 42], np.uint32)
    ki, kj = _np_split(root, 2)
    idx_i = _np_randint(ki, n_pairs, 0, n)
    idx_j = _np_randint(kj, n_pairs, 0, n)
    keep = idx_i != idx_j
    idx_i, idx_j = idx_i[keep], idx_j[keep]
    m = idx_i.shape[0]
    nch = -(-(-(-m // _NW)) // _B)
    c_tile = nch * _B
    total = c_tile * _NW
    ii = np.zeros((total,), np.int32)
    jj = np.zeros((total,), np.int32)
    ii[:m] = idx_i
    jj[:m] = idx_j
    idx = np.stack([ii.reshape(_NW, nch, _B), jj.reshape(_NW, nch, _B)], axis=2)
    out = (jnp.asarray(idx), nch, c_tile)
    _pair_cache[n] = out
    return out


def _final_kernel(lp, cp, o):
    s = jnp.sum(lp[...])
    c = jnp.sum(cp[...])
    o[0] = s / jnp.maximum(c, 1.0)


def _pack_kernel(c0, c1, c2, c3, pr, o):
    t0 = 1.0 - 0.25 * jnp.clip(c0[...], 0.0, 1.0)
    t1 = 1.0 - 0.25 * jnp.clip(c1[...], 0.0, 1.0)
    t2 = 1.0 - 0.25 * jnp.clip(c2[...], 0.0, 1.0)
    t3 = 1.0 - 0.25 * jnp.clip(c3[...], 0.0, 1.0)
    ct = 1.0 - t0 * t1 * t2 * t3
    pb = lax.bitcast_convert_type(pr[...].astype(jnp.bfloat16), jnp.uint16)
    cb = lax.bitcast_convert_type(ct.astype(jnp.bfloat16), jnp.uint16)
    word = (cb.astype(jnp.uint32) << 16) | pb.astype(jnp.uint32)
    o[...] = lax.bitcast_convert_type(word, jnp.float32)


def _make_sc_loss(n, nch, c_tile):
    mesh = plsc.VectorSubcoreMesh(core_axis_name="c", subcore_axis_name="s")

    @functools.partial(
        pl.kernel,
        out_type=[
            jax.ShapeDtypeStruct((_NW, _L), jnp.float32),
            jax.ShapeDtypeStruct((_NW, _L), jnp.float32),
        ],
        mesh=mesh,
        compiler_params=pltpu.CompilerParams(
            needs_layout_passes=False, use_tc_tiling_on_sc=False),
        scratch_types=[
            pltpu.VMEM((n,), jnp.float32),
            pltpu.VMEM((2, 2, _B), jnp.int32),
            pltpu.VMEM((2 * _L,), jnp.float32),
            pltpu.SemaphoreType.DMA((2,)),
            pltpu.SemaphoreType.DMA,
        ],
    )
    def sc_loss(tab_hbm, idx_hbm, loss_out, cnt_out, table, idxb, pout, isems,
                tsem):
        cid = lax.axis_index("c")
        sid = lax.axis_index("s")
        wid = sid * _NC + cid
        nvec = _B // _L

        in_d = [None] * nch
        in_d[0] = pltpu.async_copy(idx_hbm.at[wid, 0], idxb.at[0], isems.at[0])
        pltpu.async_copy(tab_hbm.at[pl.ds(0, n)], table, tsem).wait()
        acc = (jnp.zeros((_L,), jnp.float32), jnp.zeros((_L,), jnp.float32))
        for ch in range(nch):
            cur = ch % 2
            if ch + 1 < nch:
                in_d[ch + 1] = pltpu.async_copy(
                    idx_hbm.at[wid, ch + 1], idxb.at[1 - cur], isems.at[1 - cur])
            in_d[ch].wait()

            def p2(v, carry):
                al, ac = carry
                off = pl.multiple_of(v * _L, _L)
                ii = idxb[cur, 0, pl.ds(off, _L)]
                jj = idxb[cur, 1, pl.ds(off, _L)]
                gi = plsc.load_gather(table, [ii])
                gj = plsc.load_gather(table, [jj])
                pi, ci = plsc.unpack(plsc.bitcast(gi, jnp.bfloat16),
                                     format=plsc.PackFormat.INTERLEAVED)
                pj, cj = plsc.unpack(plsc.bitcast(gj, jnp.bfloat16),
                                     format=plsc.PackFormat.INTERLEAVED)
                pd = pi - pj
                cd = ci - cj
                sbit = plsc.bitcast(cd, jnp.int32) & jnp.int32(-2147483648)
                pdx = plsc.bitcast(plsc.bitcast(pd, jnp.int32) ^ sbit,
                                   jnp.float32)
                elem = jnp.maximum(_MARGIN - pdx, 0.0)
                mf = jnp.where(jnp.abs(cd) > 0.1, 1.0, 0.0)
                return (al + elem * mf, ac + mf)

            acc = plsc.parallel_loop(0, nvec, step=1, unroll=4, carry=acc)(p2)

        pout[pl.ds(0, _L)] = acc[0]
        pout[pl.ds(_L, _L)] = acc[1]
        pltpu.sync_copy(pout.at[pl.ds(0, _L)], loss_out.at[wid])
        pltpu.sync_copy(pout.at[pl.ds(_L, _L)], cnt_out.at[wid])

    return sc_loss


def kernel(pred_scores, cic_scores):
    pred = pred_scores.reshape(-1).astype(jnp.float32)
    n = pred.shape[0]
    idx, nch, c_tile = _pair_layout(n)

    npad = -(-n // 128) * 128
    rows = npad // 128
    cic_t = jnp.pad(cic_scores.astype(jnp.float32), ((0, npad - n), (0, 0))).T
    cols = cic_t.reshape(4, rows, 128)
    pred_rows = jnp.pad(pred, (0, npad - n)).reshape(rows, 128)
    packed = pl.pallas_call(
        _pack_kernel,
        out_shape=jax.ShapeDtypeStruct((rows, 128), jnp.float32),
    )(cols[0], cols[1], cols[2], cols[3], pred_rows).reshape(npad)

    sc_loss = _make_sc_loss(n, nch, c_tile)
    loss_part, cnt_part = sc_loss(packed, idx)

    out = pl.pallas_call(
        _final_kernel,
        out_shape=jax.ShapeDtypeStruct((1,), jnp.float32),
        out_specs=pl.BlockSpec(memory_space=pltpu.SMEM),
    )(loss_part, cnt_part)
    return out[0]
